# baseline (speedup 1.0000x reference)
import functools

import jax
import jax.numpy as jnp
from jax import lax
from jax.experimental import pallas as pl
from jax.experimental.pallas import tpu as pltpu
from jax.experimental.pallas import tpu_sc as plsc

B = 16384
L_ROWS = 100
D = 1024
N_ENC = 5

NC = 2
NS = 16
NW = NC * NS
CHUNK = 32

B_SC = 8192
B_TC = B - B_SC
TB = 512


def _merge_body(tids_ref, router_ref, enc_ref, out_ref):
    t = tids_ref[0]
    acc = router_ref[t, 0] * enc_ref[0]
    for k in range(1, N_ENC):
        acc += router_ref[t, k] * enc_ref[k]
    out_ref[...] = acc


def _merge(tids, router, enc_tables):
    return pl.pallas_call(
        _merge_body,
        out_shape=jax.ShapeDtypeStruct((L_ROWS, D), jnp.float32),
        in_specs=[
            pl.BlockSpec(memory_space=pltpu.SMEM),
            pl.BlockSpec(memory_space=pltpu.SMEM),
            pl.BlockSpec(memory_space=pltpu.VMEM),
        ],
        out_specs=pl.BlockSpec(memory_space=pltpu.VMEM),
    )(tids, router, enc_tables)


def _onehot_body(idx_ref, rw_ref, out_ref):
    ids = idx_ref[0, 0, :]
    oh = (ids[:, None] == lax.broadcasted_iota(jnp.int32, (TB, L_ROWS), 1))
    out_ref[...] = jnp.dot(oh.astype(jnp.float32), rw_ref[...],
                           preferred_element_type=jnp.float32)


def _tc_gather(idx_tc3, rw):
    nt = B_TC // TB
    return pl.pallas_call(
        _onehot_body,
        grid=(nt,),
        in_specs=[
            pl.BlockSpec((1, 1, TB), lambda i: (i, 0, 0)),
            pl.BlockSpec((L_ROWS, D), lambda i: (0, 0)),
        ],
        out_specs=pl.BlockSpec((TB, D), lambda i: (i, 0)),
        out_shape=jax.ShapeDtypeStruct((B_TC, D), jnp.float32),
    )(idx_tc3, rw)


@functools.cache
def _make_sc_gather():
    b_per_w = B_SC // NW
    nchunk = b_per_w // CHUNK
    mesh = plsc.VectorSubcoreMesh(
        core_axis_name="c", subcore_axis_name="s", num_cores=NC, num_subcores=NS
    )

    @functools.partial(
        pl.kernel,
        out_type=jax.ShapeDtypeStruct((B_SC, D), jnp.float32),
        mesh=mesh,
        scratch_types=[
            pltpu.VMEM((CHUNK,), jnp.int32),
            pltpu.VMEM((CHUNK, D), jnp.float32),
            pltpu.SemaphoreType.DMA,
        ],
    )
    def _sc_gather(idx_hbm, rw_hbm, out_hbm, idx_v, rows_v, sem):
        wid = lax.axis_index("s") * NC + lax.axis_index("c")
        base = wid * b_per_w

        def body(c, carry):
            off = base + c * CHUNK
            pltpu.sync_copy(idx_hbm.at[pl.ds(off, CHUNK)], idx_v)
            pltpu.async_copy(rw_hbm.at[idx_v], rows_v, sem).wait()
            pltpu.sync_copy(rows_v, out_hbm.at[pl.ds(off, CHUNK)])
            return carry

        lax.fori_loop(0, nchunk, body, 0)

    return _sc_gather


def kernel(prompt_token_ids, tids, router, enc_tables, input_ids):
    del input_ids
    rw = _merge(tids, router, enc_tables)
    idx = prompt_token_ids.astype(jnp.int32)
    sc_out = _make_sc_gather()(idx[:B_SC], rw)
    tc_out = _tc_gather(idx[B_SC:].reshape(B_TC // TB, 1, TB), rw)
    return jnp.concatenate([sc_out, tc_out], axis=0)


# speedup vs baseline: 1.1707x; 1.1707x over previous
import functools

import jax
import jax.numpy as jnp
from jax import lax
from jax.experimental import pallas as pl
from jax.experimental.pallas import tpu as pltpu
from jax.experimental.pallas import tpu_sc as plsc

B = 16384
L_ROWS = 100
D = 1024
N_ENC = 5

NC = 2
NS = 16
NW = NC * NS
CHUNK = 64
B_SC = B


def _merge_body(tids_ref, router_ref, enc_ref, out_ref):
    t = tids_ref[0]
    acc = router_ref[t, 0] * enc_ref[0]
    for k in range(1, N_ENC):
        acc += router_ref[t, k] * enc_ref[k]
    out_ref[...] = acc


def _merge(tids, router, enc_tables):
    return pl.pallas_call(
        _merge_body,
        out_shape=jax.ShapeDtypeStruct((L_ROWS, D), jnp.float32),
        in_specs=[
            pl.BlockSpec(memory_space=pltpu.SMEM),
            pl.BlockSpec(memory_space=pltpu.SMEM),
            pl.BlockSpec(memory_space=pltpu.VMEM),
        ],
        out_specs=pl.BlockSpec(memory_space=pltpu.VMEM),
    )(tids, router, enc_tables)


@functools.cache
def _make_sc_gather():
    b_per_w = B_SC // NW
    nchunk = b_per_w // CHUNK
    mesh = plsc.VectorSubcoreMesh(
        core_axis_name="c", subcore_axis_name="s", num_cores=NC, num_subcores=NS
    )

    @functools.partial(
        pl.kernel,
        out_type=jax.ShapeDtypeStruct((B_SC, D), jnp.float32),
        mesh=mesh,
        scratch_types=[
            pltpu.VMEM((CHUNK,), jnp.int32),
            pltpu.VMEM((CHUNK, D), jnp.float32),
            pltpu.SemaphoreType.DMA,
        ],
    )
    def _sc_gather(idx_hbm, rw_hbm, out_hbm, idx_v, rows_v, sem):
        wid = lax.axis_index("s") * NC + lax.axis_index("c")
        base = wid * b_per_w

        def body(c, carry):
            off = base + c * CHUNK
            pltpu.sync_copy(idx_hbm.at[pl.ds(off, CHUNK)], idx_v)
            pltpu.async_copy(rw_hbm.at[idx_v], rows_v, sem).wait()
            pltpu.sync_copy(rows_v, out_hbm.at[pl.ds(off, CHUNK)])
            return carry

        lax.fori_loop(0, nchunk, body, 0)

    return _sc_gather


def kernel(prompt_token_ids, tids, router, enc_tables, input_ids):
    del input_ids
    rw = _merge(tids, router, enc_tables)
    idx = prompt_token_ids.astype(jnp.int32)
    return _make_sc_gather()(idx, rw)
